# baseline (speedup 1.0000x reference)
import dataclasses
import functools
import math

import jax
import jax.numpy as jnp
from jax import lax
from jax.experimental import pallas as pl
from jax.experimental.pallas import tpu as pltpu
from jax.experimental.pallas import tpu_sc as plsc

_B, _S, _H, _D, _M, _DIN = 2, 2048, 8, 32, 32, 7
_LW = _DIN * _M
_HM = _H * _M
_NC, _NS = 2, 16
_NW = _NC * _NS
_NIDX = _B * _S * _M
_BPW = _NIDX // _NW
_SPW = _BPW // _M
_TW = _S * _DIN


_BPW1 = _S * _M // _NW
_SPW1 = _BPW1 // _M


def _sc_gather_t(table_flat, idx_flat):
    mesh = plsc.VectorSubcoreMesh(core_axis_name="c", subcore_axis_name="s")
    cp = pltpu.CompilerParams()
    if "needs_layout_passes" in pltpu.CompilerParams.__dataclass_fields__:
        cp = dataclasses.replace(cp, needs_layout_passes=False)

    @functools.partial(
        pl.kernel,
        mesh=mesh,
        compiler_params=cp,
        out_type=jax.ShapeDtypeStruct((_S * _LW,), jnp.float32),
        scratch_types=[
            pltpu.VMEM((_BPW1,), jnp.int32),
            pltpu.VMEM((_TW,), jnp.float32),
            pltpu.VMEM((_SPW1 * _LW,), jnp.float32),
        ],
    )
    def gather_kernel(table_hbm, idx_hbm, out_hbm, idx_v, tab_v, out_v):
        wid = lax.axis_index("s") * _NC + lax.axis_index("c")
        pltpu.sync_copy(idx_hbm.at[pl.ds(wid * _BPW1, _BPW1)], idx_v)
        pltpu.sync_copy(table_hbm, tab_v)

        @plsc.parallel_loop(0, _SPW1, unroll=4)
        def _(s):
            for j in range(_M // 16):
                a = idx_v[pl.ds(s * _M + j * 16, 16)] * _DIN
                for c in range(_DIN):
                    out_v[pl.ds(s * _LW + c * _M + j * 16, 16)] = (
                        plsc.load_gather(tab_v, [a + c]))

        pltpu.sync_copy(out_v, out_hbm.at[pl.ds(wid * _SPW1 * _LW,
                                                _SPW1 * _LW)])

    return gather_kernel(table_flat, idx_flat)


def _tc_body(h_ref, hgt_ref, wq_ref, wk_ref, wv_ref, wfc_ref, par_ref, out_ref):
    f32 = jnp.float32
    bf16 = jnp.bfloat16
    h7 = h_ref[...]
    hgt = hgt_ref[...]
    gamma = par_ref[0:1, :]
    beta = par_ref[1:2, :]
    bfc = par_ref[2:3, :]

    mu = jnp.sum(h7, axis=1, keepdims=True) * (1.0 / _DIN)
    xc = h7 - mu
    var = jnp.sum(xc * xc, axis=1, keepdims=True) * (1.0 / _DIN)
    qn = xc * lax.rsqrt(var + 1e-6) * gamma + beta

    wq = wq_ref[...] * (1.0 / math.sqrt(_D))
    wk = wk_ref[...]
    tdn = (((1,), (1,)), ((), ()))
    a_blocks = []
    for hh in range(_H):
        a_blocks.append(
            lax.dot_general(
                wq[:, hh * _D:(hh + 1) * _D],
                wk[:, hh * _D:(hh + 1) * _D],
                tdn,
                precision="highest",
                preferred_element_type=f32,
            )
        )
    a_cat2 = jnp.concatenate(
        [jnp.concatenate([a_blocks[hh][:, c:c + 1] for hh in range(_H)],
                         axis=1) for c in range(_DIN)], axis=1)
    qhc = jax.lax.dot(qn, a_cat2, precision="highest",
                      preferred_element_type=f32)

    wv = wv_ref[...]
    wfc = wfc_ref[...]
    g_blocks = []
    for hh in range(_H):
        g_blocks.append(
            jax.lax.dot(
                wv[:, hh * _D:(hh + 1) * _D],
                wfc[hh * _D:(hh + 1) * _D, :],
                precision="highest",
                preferred_element_type=f32,
            )
        )

    rowh = lax.broadcasted_iota(jnp.int32, (_H, _HM), 0)
    colh = lax.broadcasted_iota(jnp.int32, (_H, _HM), 1)
    eh_f = (rowh == colh // _M).astype(f32)
    rowm = lax.broadcasted_iota(jnp.int32, (_M, _HM), 0)
    colm = lax.broadcasted_iota(jnp.int32, (_M, _HM), 1)
    et_bf = (rowm == colm % _M).astype(bf16)
    ri = lax.broadcasted_iota(jnp.int32, (_HM, _HM), 0)
    ci = lax.broadcasted_iota(jnp.int32, (_HM, _HM), 1)
    tones_bf = (ri // _M == ci // _M).astype(bf16)

    hgt_bf = hgt.astype(bf16)
    hbs = []
    sc = None
    for c in range(_DIN):
        qb = jax.lax.dot(qhc[:, c * _H:(c + 1) * _H], eh_f,
                         precision="highest", preferred_element_type=f32)
        hb = jax.lax.dot(hgt_bf[:, c * _M:(c + 1) * _M], et_bf,
                         preferred_element_type=f32)
        hbs.append(hb.astype(bf16))
        t = qb * hb
        sc = t if sc is None else sc + t

    mx = jnp.max(sc, axis=1, keepdims=True)
    e = jnp.exp(sc - mx)
    gsum = jax.lax.dot(e.astype(bf16), tones_bf,
                       preferred_element_type=f32)
    w_bf = (e / gsum).astype(bf16)

    ctx = None
    for c in range(_DIN):
        tgg_rows = []
        for hh in range(_H):
            tgg_rows.append(jnp.broadcast_to(g_blocks[hh][c:c + 1, :],
                                             (_M, _DIN)))
        tgg_c = jnp.concatenate(tgg_rows, axis=0).astype(bf16)
        p = w_bf * hbs[c]
        t = jax.lax.dot(p, tgg_c, preferred_element_type=f32)
        ctx = t if ctx is None else ctx + t

    out_ref[...] = ctx + bfc + h7


def kernel(hidden_states, q_k_mask, k_q_mask, Wq, Wk, Wv, Wfc, bfc, gamma, beta):
    del k_q_mask
    f32 = jnp.float32
    idx_flat = q_k_mask.astype(jnp.int32).reshape(_S * _M)
    par = jnp.stack([gamma, beta, bfc]).astype(f32)

    tc_call = pl.pallas_call(
        _tc_body,
        grid=(1,),
        in_specs=[
            pl.BlockSpec((_S, _DIN), lambda b: (0, 0)),
            pl.BlockSpec((_S, _LW), lambda b: (0, 0)),
            pl.BlockSpec((_DIN, _H * _D), lambda b: (0, 0)),
            pl.BlockSpec((_DIN, _H * _D), lambda b: (0, 0)),
            pl.BlockSpec((_DIN, _H * _D), lambda b: (0, 0)),
            pl.BlockSpec((_H * _D, _DIN), lambda b: (0, 0)),
            pl.BlockSpec((3, _DIN), lambda b: (0, 0)),
        ],
        out_specs=pl.BlockSpec((_S, _DIN), lambda b: (0, 0)),
        out_shape=jax.ShapeDtypeStruct((_S, _DIN), f32),
    )

    outs = []
    for b in range(_B):
        h_b = hidden_states[b]
        hgt_b = _sc_gather_t(h_b.reshape(_S * _DIN), idx_flat)
        outs.append(tc_call(h_b, hgt_b.reshape(_S, _LW),
                            Wq, Wk, Wv, Wfc, par))

    return jnp.stack(outs)


# speedup vs baseline: 1.0641x; 1.0641x over previous
import dataclasses
import functools
import math

import jax
import jax.numpy as jnp
from jax import lax
from jax.experimental import pallas as pl
from jax.experimental.pallas import tpu as pltpu
from jax.experimental.pallas import tpu_sc as plsc

_B, _S, _H, _D, _M, _DIN = 2, 2048, 8, 32, 32, 7
_LW = _DIN * _M
_HM = _H * _M
_NC, _NS = 2, 16
_NW = _NC * _NS
_NIDX = _B * _S * _M
_BPW = _NIDX // _NW
_SPW = _BPW // _M
_TW = _S * _DIN


_GW = 256


def _sc_gather_t(table_flat, idx_flat):
    mesh = plsc.VectorSubcoreMesh(core_axis_name="c", subcore_axis_name="s")
    cp = pltpu.CompilerParams()
    if "needs_layout_passes" in pltpu.CompilerParams.__dataclass_fields__:
        cp = dataclasses.replace(cp, needs_layout_passes=False)

    @functools.partial(
        pl.kernel,
        mesh=mesh,
        compiler_params=cp,
        out_type=jax.ShapeDtypeStruct((_B * _S * _GW,), jnp.float32),
        scratch_types=[
            pltpu.VMEM((_BPW,), jnp.int32),
            pltpu.VMEM((_TW,), jnp.float32),
            pltpu.VMEM((_SPW * _GW,), jnp.float32),
        ],
    )
    def gather_kernel(table_hbm, idx_hbm, out_hbm, idx_v, tab_v, out_v):
        wid = lax.axis_index("s") * _NC + lax.axis_index("c")
        bat = wid // _NS
        iwin = wid % _NS
        pltpu.sync_copy(idx_hbm.at[pl.ds(iwin * _BPW, _BPW)], idx_v)
        pltpu.sync_copy(table_hbm.at[pl.ds(bat * _TW, _TW)], tab_v)

        @plsc.parallel_loop(0, _SPW, unroll=4)
        def _(s):
            for j in range(_M // 16):
                a = idx_v[pl.ds(s * _M + j * 16, 16)] * _DIN
                for c in range(_DIN):
                    out_v[pl.ds(s * _GW + c * _M + j * 16, 16)] = (
                        plsc.load_gather(tab_v, [a + c]))

        pltpu.sync_copy(out_v, out_hbm.at[pl.ds(wid * _SPW * _GW,
                                                _SPW * _GW)])

    return gather_kernel(table_flat, idx_flat)


def _tc_body(h_ref, hgt_ref, wq_ref, wk_ref, wv_ref, wfc_ref,
             gamma_ref, beta_ref, bfc_ref, out_ref):
    f32 = jnp.float32
    bf16 = jnp.bfloat16
    h7 = h_ref[...]
    hgt = hgt_ref[...]
    gamma = gamma_ref[...]
    beta = beta_ref[...]
    bfc = bfc_ref[...]

    mu = jnp.sum(h7, axis=1, keepdims=True) * (1.0 / _DIN)
    xc = h7 - mu
    var = jnp.sum(xc * xc, axis=1, keepdims=True) * (1.0 / _DIN)
    qn = xc * lax.rsqrt(var + 1e-6) * gamma + beta

    wq = wq_ref[...] * (1.0 / math.sqrt(_D))
    wk = wk_ref[...]
    tdn = (((1,), (1,)), ((), ()))
    a_blocks = []
    for hh in range(_H):
        a_blocks.append(
            lax.dot_general(
                wq[:, hh * _D:(hh + 1) * _D],
                wk[:, hh * _D:(hh + 1) * _D],
                tdn,
                precision="highest",
                preferred_element_type=f32,
            )
        )
    a_cat2 = jnp.concatenate(
        [jnp.concatenate([a_blocks[hh][:, c:c + 1] for hh in range(_H)],
                         axis=1) for c in range(_DIN)], axis=1)
    qhc = jax.lax.dot(qn, a_cat2, precision="highest",
                      preferred_element_type=f32)

    wv = wv_ref[...]
    wfc = wfc_ref[...]
    g_blocks = []
    for hh in range(_H):
        g_blocks.append(
            jax.lax.dot(
                wv[:, hh * _D:(hh + 1) * _D],
                wfc[hh * _D:(hh + 1) * _D, :],
                precision="highest",
                preferred_element_type=f32,
            )
        )

    rowh = lax.broadcasted_iota(jnp.int32, (_H, _HM), 0)
    colh = lax.broadcasted_iota(jnp.int32, (_H, _HM), 1)
    eh_f = (rowh == colh // _M).astype(f32)
    rowm = lax.broadcasted_iota(jnp.int32, (_M, _HM), 0)
    colm = lax.broadcasted_iota(jnp.int32, (_M, _HM), 1)
    et_bf = (rowm == colm % _M).astype(bf16)
    ri = lax.broadcasted_iota(jnp.int32, (_HM, _HM), 0)
    ci = lax.broadcasted_iota(jnp.int32, (_HM, _HM), 1)
    tones_bf = (ri // _M == ci // _M).astype(bf16)

    hgt_bf = hgt.astype(bf16)
    hbs = []
    sc = None
    for c in range(_DIN):
        qb = jax.lax.dot(qhc[:, c * _H:(c + 1) * _H], eh_f,
                         precision="highest", preferred_element_type=f32)
        hb = jax.lax.dot(hgt_bf[:, c * _M:(c + 1) * _M], et_bf,
                         preferred_element_type=f32)
        hbs.append(hb.astype(bf16))
        t = qb * hb
        sc = t if sc is None else sc + t

    mx = jnp.max(sc, axis=1, keepdims=True)
    e = jnp.exp(sc - mx)
    gsum = jax.lax.dot(e.astype(bf16), tones_bf,
                       preferred_element_type=f32)
    w_bf = (e / gsum).astype(bf16)

    ctx = None
    for c in range(_DIN):
        tgg_rows = []
        for hh in range(_H):
            tgg_rows.append(jnp.broadcast_to(g_blocks[hh][c:c + 1, :],
                                             (_M, _DIN)))
        tgg_c = jnp.concatenate(tgg_rows, axis=0).astype(bf16)
        p = w_bf * hbs[c]
        t = jax.lax.dot(p, tgg_c, preferred_element_type=f32)
        ctx = t if ctx is None else ctx + t

    out_ref[...] = ctx + bfc + h7


def kernel(hidden_states, q_k_mask, k_q_mask, Wq, Wk, Wv, Wfc, bfc, gamma, beta):
    del k_q_mask
    f32 = jnp.float32
    h = hidden_states.reshape(_B * _S, _DIN)
    idx_flat = q_k_mask.astype(jnp.int32).reshape(_S * _M)

    hgt_all = _sc_gather_t(hidden_states.reshape(_B * _S * _DIN), idx_flat)
    hgt2 = hgt_all.reshape(_B * _S, _GW)

    out = pl.pallas_call(
        _tc_body,
        grid=(_B,),
        in_specs=[
            pl.BlockSpec((_S, _DIN), lambda b: (b, 0)),
            pl.BlockSpec((_S, _GW), lambda b: (b, 0)),
            pl.BlockSpec((_DIN, _H * _D), lambda b: (0, 0)),
            pl.BlockSpec((_DIN, _H * _D), lambda b: (0, 0)),
            pl.BlockSpec((_DIN, _H * _D), lambda b: (0, 0)),
            pl.BlockSpec((_H * _D, _DIN), lambda b: (0, 0)),
            pl.BlockSpec((1, _DIN), lambda b: (0, 0)),
            pl.BlockSpec((1, _DIN), lambda b: (0, 0)),
            pl.BlockSpec((1, _DIN), lambda b: (0, 0)),
        ],
        out_specs=pl.BlockSpec((_S, _DIN), lambda b: (b, 0)),
        out_shape=jax.ShapeDtypeStruct((_B * _S, _DIN), f32),
    )(h, hgt2, Wq, Wk, Wv, Wfc,
      gamma.reshape(1, _DIN), beta.reshape(1, _DIN), bfc.reshape(1, _DIN))

    return out.reshape(_B, _S, _DIN)


# speedup vs baseline: 1.5534x; 1.4598x over previous
import dataclasses
import functools
import math

import jax
import jax.numpy as jnp
from jax import lax
from jax.experimental import pallas as pl
from jax.experimental.pallas import tpu as pltpu
from jax.experimental.pallas import tpu_sc as plsc

_B, _S, _H, _D, _M, _DIN = 2, 2048, 8, 32, 32, 7
_LW = _DIN * _M
_HM = _H * _M
_NC, _NS = 2, 16
_NW = _NC * _NS
_NIDX = _B * _S * _M
_BPW = _NIDX // _NW
_SPW = _BPW // _M
_TW = _S * _DIN


_GW = 256


def _sc_gather_t(table_flat, idx_flat):
    mesh = plsc.VectorSubcoreMesh(core_axis_name="c", subcore_axis_name="s")
    cp = pltpu.CompilerParams()
    if "needs_layout_passes" in pltpu.CompilerParams.__dataclass_fields__:
        cp = dataclasses.replace(cp, needs_layout_passes=False)

    @functools.partial(
        pl.kernel,
        mesh=mesh,
        compiler_params=cp,
        out_type=jax.ShapeDtypeStruct((_B * _S * _GW,), jnp.float32),
        scratch_types=[
            pltpu.VMEM((_BPW,), jnp.int32),
            pltpu.VMEM((_TW,), jnp.float32),
            pltpu.VMEM((_SPW * _GW,), jnp.float32),
        ],
    )
    def gather_kernel(table_hbm, idx_hbm, out_hbm, idx_v, tab_v, out_v):
        wid = lax.axis_index("s") * _NC + lax.axis_index("c")
        bat = wid // _NS
        iwin = wid % _NS
        pltpu.sync_copy(idx_hbm.at[pl.ds(iwin * _BPW, _BPW)], idx_v)
        pltpu.sync_copy(table_hbm.at[pl.ds(bat * _TW, _TW)], tab_v)

        @plsc.parallel_loop(0, _SPW, unroll=4)
        def _(s):
            for j in range(_M // 16):
                a = idx_v[pl.ds(s * _M + j * 16, 16)] * _DIN
                for c in range(_DIN):
                    out_v[pl.ds(s * _GW + c * _M + j * 16, 16)] = (
                        plsc.load_gather(tab_v, [a + c]))

        pltpu.sync_copy(out_v, out_hbm.at[pl.ds(wid * _SPW * _GW,
                                                _SPW * _GW)])

    return gather_kernel(table_flat, idx_flat)


def _tc_body(h_ref, hgt_ref, wq_ref, wk_ref, wv_ref, wfc_ref,
             gamma_ref, beta_ref, bfc_ref, out_ref):
    f32 = jnp.float32
    bf16 = jnp.bfloat16
    h7 = h_ref[...]
    hgt = hgt_ref[...]
    gamma = gamma_ref[...]
    beta = beta_ref[...]
    bfc = bfc_ref[...]

    mu = jnp.sum(h7, axis=1, keepdims=True) * (1.0 / _DIN)
    xc = h7 - mu
    var = jnp.sum(xc * xc, axis=1, keepdims=True) * (1.0 / _DIN)
    qn = xc * lax.rsqrt(var + 1e-6) * gamma + beta

    wq = wq_ref[...] * (1.0 / math.sqrt(_D))
    wk = wk_ref[...]
    tdn = (((1,), (1,)), ((), ()))
    a_blocks = []
    for hh in range(_H):
        a_blocks.append(
            lax.dot_general(
                wq[:, hh * _D:(hh + 1) * _D],
                wk[:, hh * _D:(hh + 1) * _D],
                tdn,
                preferred_element_type=f32,
            )
        )
    a_flat = jnp.concatenate(a_blocks, axis=1)
    pr = lax.broadcasted_iota(jnp.int32, (_H * _DIN, _H * _DIN), 0)
    pc = lax.broadcasted_iota(jnp.int32, (_H * _DIN, _H * _DIN), 1)
    pcm_bf = (pc == (pr % _DIN) * _H + pr // _DIN).astype(jnp.bfloat16)
    a_cat2 = jax.lax.dot(a_flat.astype(bf16), pcm_bf,
                         preferred_element_type=f32)
    qhc = jax.lax.dot(qn, a_cat2,
                      preferred_element_type=f32)
    qhc_bf = qhc.astype(bf16)

    wv = wv_ref[...]
    wfc = wfc_ref[...]
    tgg_blocks = []
    for hh in range(_H):
        g_h = jax.lax.dot(
            wv[:, hh * _D:(hh + 1) * _D],
            wfc[hh * _D:(hh + 1) * _D, :],
            preferred_element_type=f32,
        )
        grow = jnp.concatenate([g_h[c:c + 1, :] for c in range(_DIN)],
                               axis=1)
        tgg_blocks.append(jnp.broadcast_to(grow, (_M, _DIN * _DIN)))
    tgg = jnp.concatenate(tgg_blocks, axis=0).astype(bf16)

    rowh = lax.broadcasted_iota(jnp.int32, (_H, _HM), 0)
    colh = lax.broadcasted_iota(jnp.int32, (_H, _HM), 1)
    eh_bf = (rowh == colh // _M).astype(bf16)
    rowm = lax.broadcasted_iota(jnp.int32, (_M, _HM), 0)
    colm = lax.broadcasted_iota(jnp.int32, (_M, _HM), 1)
    et_bf = (rowm == colm % _M).astype(bf16)
    ri = lax.broadcasted_iota(jnp.int32, (_HM, _HM), 0)
    ci = lax.broadcasted_iota(jnp.int32, (_HM, _HM), 1)
    tones_bf = (ri // _M == ci // _M).astype(bf16)

    hgt_bf = hgt.astype(bf16)
    hbs = []
    sc = None
    for c in range(_DIN):
        qb = jax.lax.dot(qhc_bf[:, c * _H:(c + 1) * _H], eh_bf,
                         preferred_element_type=f32)
        hb = jax.lax.dot(hgt_bf[:, c * _M:(c + 1) * _M], et_bf,
                         preferred_element_type=f32)
        hbs.append(hb.astype(bf16))
        t = qb * hb
        sc = t if sc is None else sc + t

    mx = jnp.max(sc, axis=1, keepdims=True)
    e = jnp.exp(sc - mx)
    gsum = jax.lax.dot(e.astype(bf16), tones_bf,
                       preferred_element_type=f32)
    w_bf = (e / gsum).astype(bf16)

    ctx = None
    for c in range(_DIN):
        p = w_bf * hbs[c]
        t = jax.lax.dot(p, tgg[:, c * _DIN:(c + 1) * _DIN],
                        preferred_element_type=f32)
        ctx = t if ctx is None else ctx + t

    out_ref[...] = ctx + bfc + h7


def kernel(hidden_states, q_k_mask, k_q_mask, Wq, Wk, Wv, Wfc, bfc, gamma, beta):
    del k_q_mask
    f32 = jnp.float32
    h = hidden_states.reshape(_B * _S, _DIN)
    idx_flat = q_k_mask.astype(jnp.int32).reshape(_S * _M)

    hgt_all = _sc_gather_t(hidden_states.reshape(_B * _S * _DIN), idx_flat)
    hgt2 = hgt_all.reshape(_B * _S, _GW)

    out = pl.pallas_call(
        _tc_body,
        grid=(_B,),
        in_specs=[
            pl.BlockSpec((_S, _DIN), lambda b: (b, 0)),
            pl.BlockSpec((_S, _GW), lambda b: (b, 0)),
            pl.BlockSpec((_DIN, _H * _D), lambda b: (0, 0)),
            pl.BlockSpec((_DIN, _H * _D), lambda b: (0, 0)),
            pl.BlockSpec((_DIN, _H * _D), lambda b: (0, 0)),
            pl.BlockSpec((_H * _D, _DIN), lambda b: (0, 0)),
            pl.BlockSpec((1, _DIN), lambda b: (0, 0)),
            pl.BlockSpec((1, _DIN), lambda b: (0, 0)),
            pl.BlockSpec((1, _DIN), lambda b: (0, 0)),
        ],
        out_specs=pl.BlockSpec((_S, _DIN), lambda b: (b, 0)),
        out_shape=jax.ShapeDtypeStruct((_B * _S, _DIN), f32),
    )(h, hgt2, Wq, Wk, Wv, Wfc,
      gamma.reshape(1, _DIN), beta.reshape(1, _DIN), bfc.reshape(1, _DIN))

    return out.reshape(_B, _S, _DIN)
